# baseline (device time: 7717 ns/iter reference)
import jax
import jax.numpy as jnp
from jax import lax
from jax.experimental import pallas as pl
from jax.experimental.pallas import tpu as pltpu

N_DEV = 4


def kernel(x):
    m, n = x.shape

    def body(x_ref, out_ref, totals_ref, send_ref, send_sems, recv_sems,
             round2_sem):
        my = lax.axis_index("i")
        diag = lax.rem(my + 2, N_DEV)
        right = lax.rem(my + 1, N_DEV)
        left = lax.rem(my + 3, N_DEV)

        barrier = pltpu.get_barrier_semaphore()
        for p in (right, left):
            pl.semaphore_signal(
                barrier, inc=1, device_id=(p,),
                device_id_type=pl.DeviceIdType.MESH,
            )

        t = x_ref[:, :]
        size = m
        while size > 1:
            half = size // 2
            t = t[:half, :] * t[half:size, :]
            size = half
        send_ref[:, :] = t

        pl.semaphore_wait(barrier, 2)
        for p in (right, left):
            pl.semaphore_signal(
                round2_sem, inc=1, device_id=(p,),
                device_id_type=pl.DeviceIdType.MESH,
            )
        pl.semaphore_wait(round2_sem, 2)

        sends = []
        for k, p in enumerate((diag, right, left)):
            rdma = pltpu.make_async_remote_copy(
                src_ref=send_ref,
                dst_ref=totals_ref.at[my],
                send_sem=send_sems.at[k],
                recv_sem=recv_sems.at[my],
                device_id=(p,),
                device_id_type=pl.DeviceIdType.MESH,
            )
            rdma.start()
            sends.append(rdma)

        acc = x_ref[:, :]
        d = 1
        while d < m:
            shifted = jnp.concatenate(
                [jnp.ones((d, n), jnp.float32), acc[: m - d, :]], axis=0
            )
            acc = acc * shifted
            d *= 2

        def recv_wait(p):
            recv = pltpu.make_async_remote_copy(
                src_ref=send_ref,
                dst_ref=totals_ref.at[p],
                send_sem=send_sems.at[0],
                recv_sem=recv_sems.at[p],
                device_id=(p,),
                device_id_type=pl.DeviceIdType.MESH,
            )
            recv.wait_recv()

        def masked(p):
            return jnp.where(p < my, totals_ref[p, :, :], 1.0)

        recv_wait(right)
        recv_wait(left)
        partial = acc * (masked(right) * masked(left))
        recv_wait(diag)
        out_ref[:, :] = partial * masked(diag)

        for rdma in sends:
            rdma.wait_send()

    def body_nodiag(x_ref, out_ref, totals_ref, send_ref, send_sems, recv_sems,
                    round2_sem):
        my = lax.axis_index("i")
        right = lax.rem(my + 1, N_DEV)
        left = lax.rem(my + 3, N_DEV)
        barrier = pltpu.get_barrier_semaphore()
        for p in (right, left):
            pl.semaphore_signal(
                barrier, inc=1, device_id=(p,),
                device_id_type=pl.DeviceIdType.MESH,
            )
        t = x_ref[:, :]
        size = m
        while size > 1:
            half = size // 2
            t = t[:half, :] * t[half:size, :]
            size = half
        send_ref[:, :] = t
        pl.semaphore_wait(barrier, 2)
        sends = []
        for k, p in enumerate((right, left)):
            rdma = pltpu.make_async_remote_copy(
                src_ref=send_ref,
                dst_ref=totals_ref.at[my],
                send_sem=send_sems.at[k],
                recv_sem=recv_sems.at[my],
                device_id=(p,),
                device_id_type=pl.DeviceIdType.MESH,
            )
            rdma.start()
            sends.append(rdma)
        acc = x_ref[:, :]
        d = 1
        while d < m:
            shifted = jnp.concatenate(
                [jnp.ones((d, n), jnp.float32), acc[: m - d, :]], axis=0
            )
            acc = acc * shifted
            d *= 2
        for p in (right, left):
            recv = pltpu.make_async_remote_copy(
                src_ref=send_ref,
                dst_ref=totals_ref.at[p],
                send_sem=send_sems.at[0],
                recv_sem=recv_sems.at[p],
                device_id=(p,),
                device_id_type=pl.DeviceIdType.MESH,
            )
            recv.wait_recv()
        pr = jnp.where(right < my, totals_ref[right, :, :], 1.0)
        pl_ = jnp.where(left < my, totals_ref[left, :, :], 1.0)
        out_ref[:, :] = acc * (pr * pl_)
        for rdma in sends:
            rdma.wait_send()

    def body_empty(x_ref, out_ref, totals_ref, send_ref, send_sems, recv_sems,
                    round2_sem):
        out_ref[:, :] = x_ref[:, :]

    def body_barrier(x_ref, out_ref, totals_ref, send_ref, send_sems, recv_sems,
                    round2_sem):
        my = lax.axis_index("i")
        peers = [lax.rem(my + k, N_DEV) for k in range(1, N_DEV)]
        barrier = pltpu.get_barrier_semaphore()
        for p in peers:
            pl.semaphore_signal(
                barrier, inc=1, device_id=(p,),
                device_id_type=pl.DeviceIdType.MESH,
            )
        pl.semaphore_wait(barrier, N_DEV - 1)
        out_ref[:, :] = x_ref[:, :]

    def body_barrier2(x_ref, out_ref, totals_ref, send_ref, send_sems, recv_sems,
                    round2_sem):
        my = lax.axis_index("i")
        barrier = pltpu.get_barrier_semaphore()
        for p in (lax.rem(my + 1, N_DEV), lax.rem(my + 3, N_DEV)):
            pl.semaphore_signal(
                barrier, inc=1, device_id=(p,),
                device_id_type=pl.DeviceIdType.MESH,
            )
        pl.semaphore_wait(barrier, 2)
        out_ref[:, :] = x_ref[:, :]

    def body_barrier1(x_ref, out_ref, totals_ref, send_ref, send_sems, recv_sems,
                    round2_sem):
        my = lax.axis_index("i")
        barrier = pltpu.get_barrier_semaphore()
        pl.semaphore_signal(
            barrier, inc=1, device_id=(lax.rem(my + 1, N_DEV),),
            device_id_type=pl.DeviceIdType.MESH,
        )
        pl.semaphore_wait(barrier, 1)
        out_ref[:, :] = x_ref[:, :]

    def body_flow1(x_ref, out_ref, totals_ref, send_ref, send_sems, recv_sems,
                    round2_sem):
        my = lax.axis_index("i")
        right = lax.rem(my + 1, N_DEV)
        left = lax.rem(my + 3, N_DEV)
        barrier = pltpu.get_barrier_semaphore()
        for p in (right, left):
            pl.semaphore_signal(
                barrier, inc=1, device_id=(p,),
                device_id_type=pl.DeviceIdType.MESH,
            )
        t = x_ref[:, :]
        size = m
        while size > 1:
            half = size // 2
            t = t[:half, :] * t[half:size, :]
            size = half
        send_ref[:, :] = t
        pl.semaphore_wait(barrier, 2)
        rdma = pltpu.make_async_remote_copy(
            src_ref=send_ref,
            dst_ref=totals_ref.at[my],
            send_sem=send_sems.at[0],
            recv_sem=recv_sems.at[my],
            device_id=(right,),
            device_id_type=pl.DeviceIdType.MESH,
        )
        rdma.start()
        recv = pltpu.make_async_remote_copy(
            src_ref=send_ref,
            dst_ref=totals_ref.at[left],
            send_sem=send_sems.at[1],
            recv_sem=recv_sems.at[left],
            device_id=(left,),
            device_id_type=pl.DeviceIdType.MESH,
        )
        recv.wait_recv()
        out_ref[:, :] = x_ref[:, :] * totals_ref[left, :, :]
        rdma.wait_send()

    import os
    _probe = ""
    _probe_path = os.path.join(os.path.dirname(__file__), "probe.txt")
    if os.path.exists(_probe_path):
        _probe = open(_probe_path).read().strip()
        if _probe == "nodiag":
            body = body_nodiag
        elif _probe == "empty":
            body = body_empty
        elif _probe == "barrier":
            body = body_barrier
        elif _probe == "barrier2":
            body = body_barrier2
        elif _probe == "barrier1":
            body = body_barrier1
        elif _probe == "flow1":
            body = body_flow1

    return pl.pallas_call(
        body,
        out_shape=jax.ShapeDtypeStruct((m, n), jnp.float32),
        in_specs=[pl.BlockSpec(memory_space=pltpu.VMEM)],
        out_specs=pl.BlockSpec(memory_space=pltpu.VMEM),
        scratch_shapes=[
            pltpu.VMEM((N_DEV, 1, n), jnp.float32),
            pltpu.VMEM((1, n), jnp.float32),
            pltpu.SemaphoreType.DMA((N_DEV - 1,)),
            pltpu.SemaphoreType.DMA((N_DEV,)),
            pltpu.SemaphoreType.REGULAR,
        ],
        **(
            {}
            if _probe == "empty"
            else {"compiler_params": pltpu.CompilerParams(collective_id=0)}
        ),
    )(x)


# device time: 3922 ns/iter; 1.9676x vs baseline; 1.9676x over previous
import jax
import jax.numpy as jnp
from jax import lax
from jax.experimental import pallas as pl
from jax.experimental.pallas import tpu as pltpu

N_DEV = 4


def kernel(x):
    m, n = x.shape

    def body(x_ref, out_ref, totals_ref, send_ref, send_sems, recv_sems):
        my = lax.axis_index("i")
        diag = lax.rem(my + 2, N_DEV)
        right = lax.rem(my + 1, N_DEV)
        left = lax.rem(my + 3, N_DEV)

        barrier = pltpu.get_barrier_semaphore()
        for p in (right, diag, left):
            @pl.when(p < my)
            def _(p=p):
                pl.semaphore_signal(
                    barrier, inc=1, device_id=(p,),
                    device_id_type=pl.DeviceIdType.MESH,
                )

        t = x_ref[:, :]
        size = m
        while size > 1:
            half = size // 2
            t = t[:half, :] * t[half:size, :]
            size = half
        send_ref[:, :] = t

        pl.semaphore_wait(barrier, (N_DEV - 1) - my)

        for k, p in enumerate((diag, right, left)):
            @pl.when(p > my)
            def _(k=k, p=p):
                rdma = pltpu.make_async_remote_copy(
                    src_ref=send_ref,
                    dst_ref=totals_ref.at[my],
                    send_sem=send_sems.at[k],
                    recv_sem=recv_sems.at[my],
                    device_id=(p,),
                    device_id_type=pl.DeviceIdType.MESH,
                )
                rdma.start()

        acc = x_ref[:, :]
        d = 1
        while d < m:
            shifted = jnp.concatenate(
                [jnp.ones((d, n), jnp.float32), acc[: m - d, :]], axis=0
            )
            acc = acc * shifted
            d *= 2

        def recv_wait(p):
            @pl.when(p < my)
            def _():
                recv = pltpu.make_async_remote_copy(
                    src_ref=send_ref,
                    dst_ref=totals_ref.at[p],
                    send_sem=send_sems.at[0],
                    recv_sem=recv_sems.at[p],
                    device_id=(p,),
                    device_id_type=pl.DeviceIdType.MESH,
                )
                recv.wait_recv()

        def masked(p):
            return jnp.where(p < my, totals_ref[p, :, :], 1.0)

        recv_wait(right)
        recv_wait(left)
        partial = acc * (masked(right) * masked(left))
        recv_wait(diag)
        out_ref[:, :] = partial * masked(diag)

        for k, p in [(0, diag), (1, right), (2, left)]:
            @pl.when(p > my)
            def _(k=k, p=p):
                done = pltpu.make_async_remote_copy(
                    src_ref=send_ref,
                    dst_ref=totals_ref.at[my],
                    send_sem=send_sems.at[k],
                    recv_sem=recv_sems.at[my],
                    device_id=(p,),
                    device_id_type=pl.DeviceIdType.MESH,
                )
                done.wait_send()

    return pl.pallas_call(
        body,
        out_shape=jax.ShapeDtypeStruct((m, n), jnp.float32),
        in_specs=[pl.BlockSpec(memory_space=pltpu.VMEM)],
        out_specs=pl.BlockSpec(memory_space=pltpu.VMEM),
        scratch_shapes=[
            pltpu.VMEM((N_DEV, 1, n), jnp.float32),
            pltpu.VMEM((1, n), jnp.float32),
            pltpu.SemaphoreType.DMA((N_DEV - 1,)),
            pltpu.SemaphoreType.DMA((N_DEV,)),
        ],
        compiler_params=pltpu.CompilerParams(collective_id=0),
    )(x)
